# baseline (speedup 1.0000x reference)
import functools

import jax
import jax.numpy as jnp
from jax import lax
from jax.experimental import pallas as pl
from jax.experimental.pallas import tpu as pltpu
from jax.experimental.pallas import tpu_sc as plsc

N = 100000
E = 3200000
NC = 2
NS = 16
CH = 400
NACC = 100096
ASTRIPE = NACC // NS
CHD = 4000
NDEG = 100352
DSTRIPE = NDEG // NS
R = 4000
_ACHUNKS = [CH] * (ASTRIPE // CH) + ([ASTRIPE % CH] if ASTRIPE % CH else [])
GRID = N // R

_MESH = plsc.VectorSubcoreMesh(
    core_axis_name="c", subcore_axis_name="s", num_cores=NC, num_subcores=NS
)


def _degree_kernel(didx, ones_hbm, zeros_hbm, cnt_out,
                   ones_v, dv0, dv1, zv, deg_sp, sem_d0, sem_d1):
    c = lax.axis_index("c")
    t = lax.axis_index("s")
    pltpu.sync_copy(zeros_hbm, zv)
    pltpu.sync_copy(zv, deg_sp.at[pl.ds(t * DSTRIPE, DSTRIPE)])
    pltpu.sync_copy(ones_hbm, ones_v)
    plsc.subcore_barrier()
    base = c * (E // NC) + t * (E // NC // NS)
    nch = E // NC // NS // CHD
    npairs = nch // 2

    def ld(a, dv):
        pltpu.sync_copy(didx.at[pl.ds(base + a * CHD, CHD)], dv)

    def scat(dv, sem):
        pltpu.async_copy(ones_v, deg_sp.at[dv], sem, add=True)

    def drain(dv, sem):
        pltpu.make_async_copy(ones_v, deg_sp.at[dv], sem).wait()

    ld(0, dv0)
    scat(dv0, sem_d0)
    ld(1, dv1)
    scat(dv1, sem_d1)

    def pair(i2, carry):
        a = 2 * i2
        drain(dv0, sem_d0)
        ld(a, dv0)
        scat(dv0, sem_d0)
        drain(dv1, sem_d1)
        ld(a + 1, dv1)
        scat(dv1, sem_d1)
        return carry

    lax.fori_loop(1, npairs, pair, 0)
    drain(dv0, sem_d0)
    drain(dv1, sem_d1)
    if nch % 2:
        ld(nch - 1, dv0)
        pltpu.sync_copy(ones_v, deg_sp.at[dv0], add=True)
    plsc.subcore_barrier()
    pltpu.sync_copy(deg_sp.at[pl.ds(t * DSTRIPE, DSTRIPE)], zv)
    pltpu.sync_copy(zv, cnt_out.at[pl.ds(c * NDEG + t * DSTRIPE, DSTRIPE)])


def _spmm_kernel(table, sidx, didx, zeros_hbm, acc_out,
                 sv0, dv0, sv1, dv1, rows0, rows1, acc_sp,
                 sem_g0, sem_g1, sem_a0, sem_a1, sem_d0, sem_d1,
                 *, ept, src_core_stride, dst_core_stride, split_table):
    c = lax.axis_index("c")
    t = lax.axis_index("s")
    pltpu.sync_copy(zeros_hbm, rows0)
    r0 = t * ASTRIPE
    off = 0
    for sz in _ACHUNKS:
        pltpu.sync_copy(rows0.at[pl.ds(0, sz)], acc_sp.at[pl.ds(r0 + off, sz)])
        off += sz
    plsc.subcore_barrier()
    sbase = c * src_core_stride + t * ept
    dbase = c * dst_core_stride + t * ept
    nch = ept // CH
    npairs = nch // 2

    def idx_s(a, sv):
        pltpu.sync_copy(sidx.at[pl.ds(sbase + a * CH, CH)], sv)

    def idx_d(a, dv, sem):
        pltpu.async_copy(didx.at[pl.ds(dbase + a * CH, CH)], dv, sem)

    def wait_idx_d(a, dv, sem):
        pltpu.make_async_copy(didx.at[pl.ds(dbase + a * CH, CH)], dv, sem).wait()

    tref = table.at[c] if split_table else table

    def gather(sv, rows, sem):
        pltpu.async_copy(tref.at[sv], rows, sem)

    def wait_gather(sv, rows, sem):
        pltpu.make_async_copy(tref.at[sv], rows, sem).wait()

    def scat(rows, dv, sem):
        pltpu.async_copy(rows, acc_sp.at[dv], sem, add=True)

    def wait_scat(rows, dv, sem):
        pltpu.make_async_copy(rows, acc_sp.at[dv], sem).wait()

    idx_s(0, sv0)
    idx_d(0, dv0, sem_d0)
    gather(sv0, rows0, sem_g0)
    idx_s(1, sv1)
    gather(sv1, rows1, sem_g1)
    idx_d(1, dv1, sem_d1)
    wait_gather(sv0, rows0, sem_g0)
    wait_idx_d(0, dv0, sem_d0)
    scat(rows0, dv0, sem_a0)
    wait_gather(sv1, rows1, sem_g1)
    wait_idx_d(1, dv1, sem_d1)
    scat(rows1, dv1, sem_a1)
    wait_scat(rows0, dv0, sem_a0)

    def pair(i2, carry):
        a = 2 * i2
        idx_s(a, sv0)
        idx_d(a, dv0, sem_d0)
        gather(sv0, rows0, sem_g0)
        idx_s(a + 1, sv1)
        wait_scat(rows1, dv1, sem_a1)
        gather(sv1, rows1, sem_g1)
        idx_d(a + 1, dv1, sem_d1)
        wait_gather(sv0, rows0, sem_g0)
        wait_idx_d(a, dv0, sem_d0)
        scat(rows0, dv0, sem_a0)
        wait_gather(sv1, rows1, sem_g1)
        wait_idx_d(a + 1, dv1, sem_d1)
        scat(rows1, dv1, sem_a1)
        wait_scat(rows0, dv0, sem_a0)
        return carry

    lax.fori_loop(1, npairs, pair, 0)
    wait_scat(rows1, dv1, sem_a1)
    if nch % 2:
        a = nch - 1
        idx_s(a, sv0)
        idx_d(a, dv0, sem_d0)
        wait_idx_d(a, dv0, sem_d0)
        pltpu.sync_copy(tref.at[sv0], rows0)
        pltpu.sync_copy(rows0, acc_sp.at[dv0], add=True)
    plsc.subcore_barrier()
    off = 0
    for sz in _ACHUNKS:
        pltpu.sync_copy(acc_sp.at[pl.ds(r0 + off, sz)], rows0.at[pl.ds(0, sz)])
        pltpu.sync_copy(rows0.at[pl.ds(0, sz)],
                        acc_out.at[pl.ds(c * NACC + r0 + off, sz)])
        off += sz


def _sc_degree(d32):
    k = pl.kernel(
        _degree_kernel,
        out_type=jax.ShapeDtypeStruct((NC * NDEG,), jnp.float32),
        mesh=_MESH,
        compiler_params=pltpu.CompilerParams(use_tc_tiling_on_sc=False),
        scratch_types=[
            pltpu.VMEM((CHD,), jnp.float32),
            pltpu.VMEM((CHD,), jnp.int32),
            pltpu.VMEM((CHD,), jnp.int32),
            pltpu.VMEM((DSTRIPE,), jnp.float32),
            pltpu.VMEM_SHARED((NDEG,), jnp.float32),
            pltpu.SemaphoreType.DMA,
            pltpu.SemaphoreType.DMA,
        ],
    )
    return k(d32, jnp.ones((CHD,), jnp.float32), jnp.zeros((DSTRIPE,), jnp.float32))


def _sc_spmm(table, sidx, didx, *, ept, src_core_stride, dst_core_stride,
             split_table=False):
    body = functools.partial(
        _spmm_kernel, ept=ept,
        src_core_stride=src_core_stride, dst_core_stride=dst_core_stride,
        split_table=split_table,
    )
    k = pl.kernel(
        body,
        out_type=jax.ShapeDtypeStruct((NC * NACC, 16), jnp.float32),
        mesh=_MESH,
        compiler_params=pltpu.CompilerParams(use_tc_tiling_on_sc=False),
        scratch_types=[
            pltpu.VMEM((CH,), jnp.int32),
            pltpu.VMEM((CH,), jnp.int32),
            pltpu.VMEM((CH,), jnp.int32),
            pltpu.VMEM((CH,), jnp.int32),
            pltpu.VMEM((CH, 16), jnp.float32),
            pltpu.VMEM((CH, 16), jnp.float32),
            pltpu.VMEM_SHARED((NACC, 16), jnp.float32),
            pltpu.SemaphoreType.DMA,
            pltpu.SemaphoreType.DMA,
            pltpu.SemaphoreType.DMA,
            pltpu.SemaphoreType.DMA,
            pltpu.SemaphoreType.DMA,
            pltpu.SemaphoreType.DMA,
        ],
    )
    return k(table, sidx, didx, jnp.zeros((CH, 16), jnp.float32))


def _mm1_body(x_ref, w1_ref, h_ref):
    h = jnp.dot(x_ref[...], w1_ref[...], preferred_element_type=jnp.float32)
    h_ref[0] = h[:, :16]
    h_ref[1] = h[:, 16:]


def _scale_body(h_ref, cnt_ref, hs_ref):
    cb = cnt_ref[...]
    dis = lax.rsqrt(cb[0, :, 0] + cb[1, :, 0] + 1.0)
    hs_ref[0] = h_ref[0] * dis[:, None]
    hs_ref[1] = h_ref[1] * dis[:, None]


def _mid_body(acc_ref, hs_ref, cnt_ref, w2_ref, b1_ref, hs2_ref):
    cb = cnt_ref[...]
    dis = lax.rsqrt(cb[0, :, 0] + cb[1, :, 0] + 1.0)
    a = jnp.concatenate(
        [acc_ref[0] + hs_ref[0], acc_ref[1] + hs_ref[1]], axis=1)
    out1 = jnp.maximum(a * dis[:, None] + b1_ref[0], 0.0)
    hs2 = jnp.dot(out1, w2_ref[...], preferred_element_type=jnp.float32)
    hs2_ref[...] = hs2 * dis[:, None]


def _out_body(acc_ref, hs2_ref, cnt_ref, b2_ref, o_ref):
    cb = cnt_ref[...]
    dis = lax.rsqrt(cb[0, :, 0] + cb[1, :, 0] + 1.0)
    o_ref[...] = (acc_ref[0] + acc_ref[1] + hs2_ref[...]) * dis[:, None] + b2_ref[0]


def _rowspec(shape3):
    return pl.BlockSpec(shape3, lambda i: (0, i, 0))


_CNT_SPEC = pl.BlockSpec((2, R, 1), lambda i: (0, i, 0))
_FULL2 = lambda a, b: pl.BlockSpec((a, b), lambda i: (0, 0))


def _tc_mm1(x, W1):
    return pl.pallas_call(
        _mm1_body,
        grid=(GRID,),
        in_specs=[
            pl.BlockSpec((R, 10), lambda i: (i, 0)),
            _FULL2(10, 32),
        ],
        out_specs=_rowspec((2, R, 16)),
        out_shape=jax.ShapeDtypeStruct((2, N, 16), jnp.float32),
    )(x, W1)


def _tc_scale(h3, cnt3):
    return pl.pallas_call(
        _scale_body,
        grid=(GRID,),
        in_specs=[
            _rowspec((2, R, 16)),
            _CNT_SPEC,
        ],
        out_specs=_rowspec((2, R, 16)),
        out_shape=jax.ShapeDtypeStruct((2, N, 16), jnp.float32),
    )(h3, cnt3)


def _tc_mid(acc3, hs3, cnt3, W2, b1):
    return pl.pallas_call(
        _mid_body,
        grid=(GRID,),
        in_specs=[
            _rowspec((2, R, 16)),
            _rowspec((2, R, 16)),
            _CNT_SPEC,
            _FULL2(32, 16),
            _FULL2(1, 32),
        ],
        out_specs=pl.BlockSpec((R, 16), lambda i: (i, 0)),
        out_shape=jax.ShapeDtypeStruct((N, 16), jnp.float32),
    )(acc3, hs3, cnt3, W2, b1)


def _tc_out(acc3, hs2, cnt3, b2):
    return pl.pallas_call(
        _out_body,
        grid=(GRID,),
        in_specs=[
            _rowspec((2, R, 16)),
            pl.BlockSpec((R, 16), lambda i: (i, 0)),
            _CNT_SPEC,
            _FULL2(1, 16),
        ],
        out_specs=pl.BlockSpec((R, 16), lambda i: (i, 0)),
        out_shape=jax.ShapeDtypeStruct((N, 16), jnp.float32),
    )(acc3, hs2, cnt3, b2)


def kernel(x, edge_index, W1, b1, W2, b2):
    s = edge_index[0].astype(jnp.int32)
    d = edge_index[1].astype(jnp.int32)

    h3 = _tc_mm1(x, W1)
    cnt = _sc_degree(d)
    cnt3 = cnt.reshape(NC, NDEG, 1)

    hs1 = _tc_scale(h3, cnt3)
    acc1 = _sc_spmm(
        hs1, s, d,
        ept=E // NS, src_core_stride=0, dst_core_stride=0,
        split_table=True,
    ).reshape(NC, NACC, 16)

    hs2 = _tc_mid(acc1, hs1, cnt3, W2, b1.reshape(1, 32))
    acc2 = _sc_spmm(
        hs2, s, d,
        ept=E // NC // NS, src_core_stride=E // NC, dst_core_stride=E // NC,
    ).reshape(NC, NACC, 16)

    return _tc_out(acc2, hs2, cnt3, b2.reshape(1, 16))


# speedup vs baseline: 1.1327x; 1.1327x over previous
import functools

import jax
import jax.numpy as jnp
from jax import lax
from jax.experimental import pallas as pl
from jax.experimental.pallas import tpu as pltpu
from jax.experimental.pallas import tpu_sc as plsc

N = 100000
E = 3200000
NC = 2
NS = 16
CH = 800
NACC = 100096
ASTRIPE = NACC // NS
CHD = 4000
NDEG = 100352
DSTRIPE = NDEG // NS
R = 4000
_ACHUNKS = [CH] * (ASTRIPE // CH) + ([ASTRIPE % CH] if ASTRIPE % CH else [])
GRID = N // R

_MESH = plsc.VectorSubcoreMesh(
    core_axis_name="c", subcore_axis_name="s", num_cores=NC, num_subcores=NS
)


def _degree_kernel(didx, ones_hbm, zeros_hbm, cnt_out,
                   ones_v, dv0, dv1, zv, deg_sp, sem_d0, sem_d1):
    c = lax.axis_index("c")
    t = lax.axis_index("s")
    pltpu.sync_copy(zeros_hbm, zv)
    pltpu.sync_copy(zv, deg_sp.at[pl.ds(t * DSTRIPE, DSTRIPE)])
    pltpu.sync_copy(ones_hbm, ones_v)
    plsc.subcore_barrier()
    base = c * (E // NC) + t * (E // NC // NS)
    nch = E // NC // NS // CHD
    npairs = nch // 2

    def ld(a, dv):
        pltpu.sync_copy(didx.at[pl.ds(base + a * CHD, CHD)], dv)

    def scat(dv, sem):
        pltpu.async_copy(ones_v, deg_sp.at[dv], sem, add=True)

    def drain(dv, sem):
        pltpu.make_async_copy(ones_v, deg_sp.at[dv], sem).wait()

    ld(0, dv0)
    scat(dv0, sem_d0)
    ld(1, dv1)
    scat(dv1, sem_d1)

    def pair(i2, carry):
        a = 2 * i2
        drain(dv0, sem_d0)
        ld(a, dv0)
        scat(dv0, sem_d0)
        drain(dv1, sem_d1)
        ld(a + 1, dv1)
        scat(dv1, sem_d1)
        return carry

    lax.fori_loop(1, npairs, pair, 0)
    drain(dv0, sem_d0)
    drain(dv1, sem_d1)
    if nch % 2:
        ld(nch - 1, dv0)
        pltpu.sync_copy(ones_v, deg_sp.at[dv0], add=True)
    plsc.subcore_barrier()
    pltpu.sync_copy(deg_sp.at[pl.ds(t * DSTRIPE, DSTRIPE)], zv)
    pltpu.sync_copy(zv, cnt_out.at[pl.ds(c * NDEG + t * DSTRIPE, DSTRIPE)])


def _spmm_kernel(table, sidx, didx2, zeros_hbm, acc_out,
                 sv_big, dv_big, rows0, rows1, acc_sp,
                 sem_g0, sem_g1, sem_a0, sem_a1, sem_d,
                 *, ept, src_core_stride, dst_core_stride, split_table):
    c = lax.axis_index("c")
    t = lax.axis_index("s")
    pltpu.sync_copy(zeros_hbm, rows0)
    r0 = t * ASTRIPE
    off = 0
    for sz in _ACHUNKS:
        pltpu.sync_copy(rows0.at[pl.ds(0, sz)], acc_sp.at[pl.ds(r0 + off, sz)])
        off += sz
    plsc.subcore_barrier()
    sbase = c * src_core_stride + t * ept
    drow = (c * dst_core_stride + t * ept) // CH
    nch = ept // CH
    npairs = nch // 2

    tref = table.at[c] if split_table else table

    def idx_s2(a):
        pltpu.sync_copy(sidx.at[pl.ds(sbase + a * CH, 2 * CH)], sv_big)

    def idx_d2(a):
        pltpu.async_copy(didx2.at[pl.ds(drow + a, 2)], dv_big, sem_d)

    def wait_idx_d2(a):
        pltpu.make_async_copy(didx2.at[pl.ds(drow + a, 2)], dv_big, sem_d).wait()

    def gather(k, rows, sem):
        pltpu.async_copy(tref.at[sv_big.at[pl.ds(k * CH, CH)]], rows, sem)

    def wait_gather(k, rows, sem):
        pltpu.make_async_copy(
            tref.at[sv_big.at[pl.ds(k * CH, CH)]], rows, sem).wait()

    def scat(rows, k, sem):
        pltpu.async_copy(rows, acc_sp.at[dv_big.at[k]], sem, add=True)

    def wait_scat(rows, k, sem):
        pltpu.make_async_copy(rows, acc_sp.at[dv_big.at[k]], sem).wait()

    idx_s2(0)
    idx_d2(0)
    gather(0, rows0, sem_g0)
    gather(1, rows1, sem_g1)
    wait_gather(0, rows0, sem_g0)
    wait_idx_d2(0)
    scat(rows0, 0, sem_a0)
    wait_gather(1, rows1, sem_g1)
    scat(rows1, 1, sem_a1)
    wait_scat(rows0, 0, sem_a0)

    def pair(i2, carry):
        a = 2 * i2
        idx_s2(a)
        gather(0, rows0, sem_g0)
        wait_scat(rows1, 1, sem_a1)
        idx_d2(a)
        gather(1, rows1, sem_g1)
        wait_gather(0, rows0, sem_g0)
        wait_idx_d2(a)
        scat(rows0, 0, sem_a0)
        wait_gather(1, rows1, sem_g1)
        scat(rows1, 1, sem_a1)
        wait_scat(rows0, 0, sem_a0)
        return carry

    lax.fori_loop(1, npairs, pair, 0)
    wait_scat(rows1, 1, sem_a1)
    if nch % 2:
        a = nch - 1
        pltpu.sync_copy(sidx.at[pl.ds(sbase + a * CH, CH)],
                        sv_big.at[pl.ds(0, CH)])
        pltpu.sync_copy(didx2.at[pl.ds(drow + a, 1)],
                        dv_big.at[pl.ds(0, 1)])
        pltpu.sync_copy(tref.at[sv_big.at[pl.ds(0, CH)]], rows0)
        pltpu.sync_copy(rows0, acc_sp.at[dv_big.at[0]], add=True)
    plsc.subcore_barrier()
    off = 0
    for sz in _ACHUNKS:
        pltpu.sync_copy(acc_sp.at[pl.ds(r0 + off, sz)], rows0.at[pl.ds(0, sz)])
        pltpu.sync_copy(rows0.at[pl.ds(0, sz)],
                        acc_out.at[pl.ds(c * NACC + r0 + off, sz)])
        off += sz


def _sc_degree(d32):
    k = pl.kernel(
        _degree_kernel,
        out_type=jax.ShapeDtypeStruct((NC * NDEG,), jnp.float32),
        mesh=_MESH,
        compiler_params=pltpu.CompilerParams(use_tc_tiling_on_sc=False),
        scratch_types=[
            pltpu.VMEM((CHD,), jnp.float32),
            pltpu.VMEM((CHD,), jnp.int32),
            pltpu.VMEM((CHD,), jnp.int32),
            pltpu.VMEM((DSTRIPE,), jnp.float32),
            pltpu.VMEM_SHARED((NDEG,), jnp.float32),
            pltpu.SemaphoreType.DMA,
            pltpu.SemaphoreType.DMA,
        ],
    )
    return k(d32, jnp.ones((CHD,), jnp.float32), jnp.zeros((DSTRIPE,), jnp.float32))


def _sc_spmm(table, sidx, didx, *, ept, src_core_stride, dst_core_stride,
             split_table=False):
    body = functools.partial(
        _spmm_kernel, ept=ept,
        src_core_stride=src_core_stride, dst_core_stride=dst_core_stride,
        split_table=split_table,
    )
    k = pl.kernel(
        body,
        out_type=jax.ShapeDtypeStruct((NC * NACC, 16), jnp.float32),
        mesh=_MESH,
        compiler_params=pltpu.CompilerParams(use_tc_tiling_on_sc=False),
        scratch_types=[
            pltpu.VMEM((2 * CH,), jnp.int32),
            pltpu.VMEM((2, CH), jnp.int32),
            pltpu.VMEM((CH, 16), jnp.float32),
            pltpu.VMEM((CH, 16), jnp.float32),
            pltpu.VMEM_SHARED((NACC, 16), jnp.float32),
            pltpu.SemaphoreType.DMA,
            pltpu.SemaphoreType.DMA,
            pltpu.SemaphoreType.DMA,
            pltpu.SemaphoreType.DMA,
            pltpu.SemaphoreType.DMA,
        ],
    )
    return k(table, sidx, didx.reshape(didx.shape[0] // CH, CH),
             jnp.zeros((CH, 16), jnp.float32))


def _mm1_body(x_ref, w1_ref, h_ref):
    h = jnp.dot(x_ref[...], w1_ref[...], preferred_element_type=jnp.float32)
    h_ref[0] = h[:, :16]
    h_ref[1] = h[:, 16:]


def _scale_body(h_ref, cnt_ref, hs_ref):
    cb = cnt_ref[...]
    dis = lax.rsqrt(cb[0, :, 0] + cb[1, :, 0] + 1.0)
    hs_ref[0] = h_ref[0] * dis[:, None]
    hs_ref[1] = h_ref[1] * dis[:, None]


def _mid_body(acc_ref, hs_ref, cnt_ref, w2_ref, b1_ref, hs2_ref):
    cb = cnt_ref[...]
    dis = lax.rsqrt(cb[0, :, 0] + cb[1, :, 0] + 1.0)
    a = jnp.concatenate(
        [acc_ref[0] + hs_ref[0], acc_ref[1] + hs_ref[1]], axis=1)
    out1 = jnp.maximum(a * dis[:, None] + b1_ref[0], 0.0)
    hs2 = jnp.dot(out1, w2_ref[...], preferred_element_type=jnp.float32)
    hs2_ref[...] = hs2 * dis[:, None]


def _out_body(acc_ref, hs2_ref, cnt_ref, b2_ref, o_ref):
    cb = cnt_ref[...]
    dis = lax.rsqrt(cb[0, :, 0] + cb[1, :, 0] + 1.0)
    o_ref[...] = (acc_ref[0] + acc_ref[1] + hs2_ref[...]) * dis[:, None] + b2_ref[0]


def _rowspec(shape3):
    return pl.BlockSpec(shape3, lambda i: (0, i, 0))


_CNT_SPEC = pl.BlockSpec((2, R, 1), lambda i: (0, i, 0))
_FULL2 = lambda a, b: pl.BlockSpec((a, b), lambda i: (0, 0))


def _tc_mm1(x, W1):
    return pl.pallas_call(
        _mm1_body,
        grid=(GRID,),
        in_specs=[
            pl.BlockSpec((R, 10), lambda i: (i, 0)),
            _FULL2(10, 32),
        ],
        out_specs=_rowspec((2, R, 16)),
        out_shape=jax.ShapeDtypeStruct((2, N, 16), jnp.float32),
    )(x, W1)


def _tc_scale(h3, cnt3):
    return pl.pallas_call(
        _scale_body,
        grid=(GRID,),
        in_specs=[
            _rowspec((2, R, 16)),
            _CNT_SPEC,
        ],
        out_specs=_rowspec((2, R, 16)),
        out_shape=jax.ShapeDtypeStruct((2, N, 16), jnp.float32),
    )(h3, cnt3)


def _tc_mid(acc3, hs3, cnt3, W2, b1):
    return pl.pallas_call(
        _mid_body,
        grid=(GRID,),
        in_specs=[
            _rowspec((2, R, 16)),
            _rowspec((2, R, 16)),
            _CNT_SPEC,
            _FULL2(32, 16),
            _FULL2(1, 32),
        ],
        out_specs=pl.BlockSpec((R, 16), lambda i: (i, 0)),
        out_shape=jax.ShapeDtypeStruct((N, 16), jnp.float32),
    )(acc3, hs3, cnt3, W2, b1)


def _tc_out(acc3, hs2, cnt3, b2):
    return pl.pallas_call(
        _out_body,
        grid=(GRID,),
        in_specs=[
            _rowspec((2, R, 16)),
            pl.BlockSpec((R, 16), lambda i: (i, 0)),
            _CNT_SPEC,
            _FULL2(1, 16),
        ],
        out_specs=pl.BlockSpec((R, 16), lambda i: (i, 0)),
        out_shape=jax.ShapeDtypeStruct((N, 16), jnp.float32),
    )(acc3, hs2, cnt3, b2)


def kernel(x, edge_index, W1, b1, W2, b2):
    s = edge_index[0].astype(jnp.int32)
    d = edge_index[1].astype(jnp.int32)

    h3 = _tc_mm1(x, W1)
    cnt = _sc_degree(d)
    cnt3 = cnt.reshape(NC, NDEG, 1)

    hs1 = _tc_scale(h3, cnt3)
    acc1 = _sc_spmm(
        hs1, s, d,
        ept=E // NS, src_core_stride=0, dst_core_stride=0,
        split_table=True,
    ).reshape(NC, NACC, 16)

    hs2 = _tc_mid(acc1, hs1, cnt3, W2, b1.reshape(1, 32))
    acc2 = _sc_spmm(
        hs2, s, d,
        ept=E // NC // NS, src_core_stride=E // NC, dst_core_stride=E // NC,
    ).reshape(NC, NACC, 16)

    return _tc_out(acc2, hs2, cnt3, b2.reshape(1, 16))


# speedup vs baseline: 1.5461x; 1.3650x over previous
import functools

import jax
import jax.numpy as jnp
from jax import lax
from jax.experimental import pallas as pl
from jax.experimental.pallas import tpu as pltpu
from jax.experimental.pallas import tpu_sc as plsc

N = 100000
E = 3200000
NC = 2
NS = 16
CH = 800
NACC = 100096
ASTRIPE = NACC // NS
CHD = 4000
NDEG = 100352
DSTRIPE = NDEG // NS
R = 4000


def _achunks(ch):
    return [ch] * (ASTRIPE // ch) + ([ASTRIPE % ch] if ASTRIPE % ch else [])
GRID = N // R

_MESH = plsc.VectorSubcoreMesh(
    core_axis_name="c", subcore_axis_name="s", num_cores=NC, num_subcores=NS
)


def _degree_kernel(didx, ones_hbm, zeros_hbm, cnt_out,
                   ones_v, dv0, dv1, zv, deg_sp, sem_d0, sem_d1):
    c = lax.axis_index("c")
    t = lax.axis_index("s")
    pltpu.sync_copy(zeros_hbm, zv)
    pltpu.sync_copy(zv, deg_sp.at[pl.ds(t * DSTRIPE, DSTRIPE)])
    pltpu.sync_copy(ones_hbm, ones_v)
    plsc.subcore_barrier()
    base = c * (E // NC) + t * (E // NC // NS)
    nch = E // NC // NS // CHD
    npairs = nch // 2

    def ld(a, dv):
        pltpu.sync_copy(didx.at[pl.ds(base + a * CHD, CHD)], dv)

    def scat(dv, sem):
        pltpu.async_copy(ones_v, deg_sp.at[dv], sem, add=True)

    def drain(dv, sem):
        pltpu.make_async_copy(ones_v, deg_sp.at[dv], sem).wait()

    ld(0, dv0)
    scat(dv0, sem_d0)
    ld(1, dv1)
    scat(dv1, sem_d1)

    def pair(i2, carry):
        a = 2 * i2
        drain(dv0, sem_d0)
        ld(a, dv0)
        scat(dv0, sem_d0)
        drain(dv1, sem_d1)
        ld(a + 1, dv1)
        scat(dv1, sem_d1)
        return carry

    lax.fori_loop(1, npairs, pair, 0)
    drain(dv0, sem_d0)
    drain(dv1, sem_d1)
    if nch % 2:
        ld(nch - 1, dv0)
        pltpu.sync_copy(ones_v, deg_sp.at[dv0], add=True)
    plsc.subcore_barrier()
    pltpu.sync_copy(deg_sp.at[pl.ds(t * DSTRIPE, DSTRIPE)], zv)
    pltpu.sync_copy(zv, cnt_out.at[pl.ds(c * NDEG + t * DSTRIPE, DSTRIPE)])


def _spmm_kernel(table, sidx, didx, zeros_hbm, acc_out,
                 sv0, dv0, sv1, dv1, rows0, rows1, acc_sp,
                 sem_g0, sem_g1, sem_a0, sem_a1, sem_d0, sem_d1,
                 *, ept, src_core_stride, dst_core_stride, split_table, ch):
    c = lax.axis_index("c")
    t = lax.axis_index("s")
    pltpu.sync_copy(zeros_hbm, rows0)
    r0 = t * ASTRIPE
    off = 0
    for sz in _achunks(ch):
        pltpu.sync_copy(rows0.at[pl.ds(0, sz)], acc_sp.at[pl.ds(r0 + off, sz)])
        off += sz
    plsc.subcore_barrier()
    sbase = c * src_core_stride + t * ept
    dbase = c * dst_core_stride + t * ept
    nch = ept // ch
    npairs = nch // 2

    def idx_s(a, sv):
        pltpu.sync_copy(sidx.at[pl.ds(sbase + a * ch, ch)], sv)

    def idx_d(a, dv, sem):
        pltpu.async_copy(didx.at[pl.ds(dbase + a * ch, ch)], dv, sem)

    def wait_idx_d(a, dv, sem):
        pltpu.make_async_copy(didx.at[pl.ds(dbase + a * ch, ch)], dv, sem).wait()

    tref = table.at[c] if split_table else table

    def gather(sv, rows, sem):
        pltpu.async_copy(tref.at[sv], rows, sem)

    def wait_gather(sv, rows, sem):
        pltpu.make_async_copy(tref.at[sv], rows, sem).wait()

    def scat(rows, dv, sem):
        pltpu.async_copy(rows, acc_sp.at[dv], sem, add=True)

    def wait_scat(rows, dv, sem):
        pltpu.make_async_copy(rows, acc_sp.at[dv], sem).wait()

    idx_s(0, sv0)
    idx_d(0, dv0, sem_d0)
    gather(sv0, rows0, sem_g0)
    idx_s(1, sv1)
    gather(sv1, rows1, sem_g1)
    idx_d(1, dv1, sem_d1)
    wait_gather(sv0, rows0, sem_g0)
    wait_idx_d(0, dv0, sem_d0)
    scat(rows0, dv0, sem_a0)
    wait_gather(sv1, rows1, sem_g1)
    wait_idx_d(1, dv1, sem_d1)
    scat(rows1, dv1, sem_a1)
    wait_scat(rows0, dv0, sem_a0)

    def pair(i2, carry):
        a = 2 * i2
        idx_s(a, sv0)
        idx_d(a, dv0, sem_d0)
        gather(sv0, rows0, sem_g0)
        idx_s(a + 1, sv1)
        wait_scat(rows1, dv1, sem_a1)
        gather(sv1, rows1, sem_g1)
        idx_d(a + 1, dv1, sem_d1)
        wait_gather(sv0, rows0, sem_g0)
        wait_idx_d(a, dv0, sem_d0)
        scat(rows0, dv0, sem_a0)
        wait_gather(sv1, rows1, sem_g1)
        wait_idx_d(a + 1, dv1, sem_d1)
        scat(rows1, dv1, sem_a1)
        wait_scat(rows0, dv0, sem_a0)
        return carry

    lax.fori_loop(1, npairs, pair, 0)
    wait_scat(rows1, dv1, sem_a1)
    if nch % 2:
        a = nch - 1
        idx_s(a, sv0)
        idx_d(a, dv0, sem_d0)
        wait_idx_d(a, dv0, sem_d0)
        pltpu.sync_copy(tref.at[sv0], rows0)
        pltpu.sync_copy(rows0, acc_sp.at[dv0], add=True)
    plsc.subcore_barrier()
    off = 0
    for sz in _achunks(ch):
        pltpu.sync_copy(acc_sp.at[pl.ds(r0 + off, sz)], rows0.at[pl.ds(0, sz)])
        pltpu.sync_copy(rows0.at[pl.ds(0, sz)],
                        acc_out.at[pl.ds(c * NACC + r0 + off, sz)])
        off += sz


def _sc_degree(d32):
    k = pl.kernel(
        _degree_kernel,
        out_type=jax.ShapeDtypeStruct((NC * NDEG,), jnp.float32),
        mesh=_MESH,
        compiler_params=pltpu.CompilerParams(use_tc_tiling_on_sc=False),
        scratch_types=[
            pltpu.VMEM((CHD,), jnp.float32),
            pltpu.VMEM((CHD,), jnp.int32),
            pltpu.VMEM((CHD,), jnp.int32),
            pltpu.VMEM((DSTRIPE,), jnp.float32),
            pltpu.VMEM_SHARED((NDEG,), jnp.float32),
            pltpu.SemaphoreType.DMA,
            pltpu.SemaphoreType.DMA,
        ],
    )
    return k(d32, jnp.ones((CHD,), jnp.float32), jnp.zeros((DSTRIPE,), jnp.float32))


def _sc_spmm(table, sidx, didx, *, ept, src_core_stride, dst_core_stride,
             split_table=False, width=16, ch=CH):
    body = functools.partial(
        _spmm_kernel, ept=ept,
        src_core_stride=src_core_stride, dst_core_stride=dst_core_stride,
        split_table=split_table, ch=ch,
    )
    k = pl.kernel(
        body,
        out_type=jax.ShapeDtypeStruct((NC * NACC, width), jnp.float32),
        mesh=_MESH,
        compiler_params=pltpu.CompilerParams(use_tc_tiling_on_sc=False),
        scratch_types=[
            pltpu.VMEM((ch,), jnp.int32),
            pltpu.VMEM((ch,), jnp.int32),
            pltpu.VMEM((ch,), jnp.int32),
            pltpu.VMEM((ch,), jnp.int32),
            pltpu.VMEM((ch, width), jnp.float32),
            pltpu.VMEM((ch, width), jnp.float32),
            pltpu.VMEM_SHARED((NACC, width), jnp.float32),
            pltpu.SemaphoreType.DMA,
            pltpu.SemaphoreType.DMA,
            pltpu.SemaphoreType.DMA,
            pltpu.SemaphoreType.DMA,
            pltpu.SemaphoreType.DMA,
            pltpu.SemaphoreType.DMA,
        ],
    )
    return k(table, sidx, didx, jnp.zeros((ch, width), jnp.float32))


def _xs_body(x_ref, cnt_ref, xs_ref):
    cb = cnt_ref[...]
    dis = lax.rsqrt(cb[0, :, 0] + cb[1, :, 0] + 1.0)
    xs = x_ref[...] * dis[:, None]
    xs_ref[...] = jnp.concatenate(
        [xs, jnp.zeros((xs.shape[0], 6), jnp.float32)], axis=1)


def _mid_body(acc_ref, xs_ref, cnt_ref, w1_ref, b1_ref, w2_ref, hs2_ref):
    cb = cnt_ref[...]
    dis = lax.rsqrt(cb[0, :, 0] + cb[1, :, 0] + 1.0)
    ax = (acc_ref[0] + acc_ref[1] + xs_ref[...]) * dis[:, None]
    out1 = jnp.maximum(
        jnp.dot(ax, w1_ref[...], preferred_element_type=jnp.float32)
        + b1_ref[0], 0.0)
    hs2 = jnp.dot(out1, w2_ref[...], preferred_element_type=jnp.float32)
    hs2_ref[...] = hs2 * dis[:, None]


def _out_body(acc_ref, hs2_ref, cnt_ref, b2_ref, o_ref):
    cb = cnt_ref[...]
    dis = lax.rsqrt(cb[0, :, 0] + cb[1, :, 0] + 1.0)
    o_ref[...] = (acc_ref[0] + acc_ref[1] + hs2_ref[...]) * dis[:, None] + b2_ref[0]


def _rowspec(shape3):
    return pl.BlockSpec(shape3, lambda i: (0, i, 0))


_CNT_SPEC = pl.BlockSpec((2, R, 1), lambda i: (0, i, 0))
_FULL2 = lambda a, b: pl.BlockSpec((a, b), lambda i: (0, 0))


def _tc_xs(x, cnt3):
    return pl.pallas_call(
        _xs_body,
        grid=(GRID,),
        in_specs=[
            pl.BlockSpec((R, 10), lambda i: (i, 0)),
            _CNT_SPEC,
        ],
        out_specs=pl.BlockSpec((R, 16), lambda i: (i, 0)),
        out_shape=jax.ShapeDtypeStruct((N, 16), jnp.float32),
    )(x, cnt3)


def _tc_mid(acc3, xs, cnt3, W1, b1, W2):
    return pl.pallas_call(
        _mid_body,
        grid=(GRID,),
        in_specs=[
            _rowspec((2, R, 16)),
            pl.BlockSpec((R, 16), lambda i: (i, 0)),
            _CNT_SPEC,
            _FULL2(16, 32),
            _FULL2(1, 32),
            _FULL2(32, 16),
        ],
        out_specs=pl.BlockSpec((R, 16), lambda i: (i, 0)),
        out_shape=jax.ShapeDtypeStruct((N, 16), jnp.float32),
    )(acc3, xs, cnt3, W1, b1, W2)


def _tc_out(acc3, hs2, cnt3, b2):
    return pl.pallas_call(
        _out_body,
        grid=(GRID,),
        in_specs=[
            _rowspec((2, R, 16)),
            pl.BlockSpec((R, 16), lambda i: (i, 0)),
            _CNT_SPEC,
            _FULL2(1, 16),
        ],
        out_specs=pl.BlockSpec((R, 16), lambda i: (i, 0)),
        out_shape=jax.ShapeDtypeStruct((N, 16), jnp.float32),
    )(acc3, hs2, cnt3, b2)


def kernel(x, edge_index, W1, b1, W2, b2):
    s = edge_index[0].astype(jnp.int32)
    d = edge_index[1].astype(jnp.int32)

    cnt = _sc_degree(d)
    cnt3 = cnt.reshape(NC, NDEG, 1)

    xs = _tc_xs(x, cnt3)
    acc1 = _sc_spmm(
        xs, s, d,
        ept=E // NC // NS, src_core_stride=E // NC, dst_core_stride=E // NC,
        width=16, ch=CH,
    ).reshape(NC, NACC, 16)

    W1p = jnp.concatenate([W1, jnp.zeros((6, 32), W1.dtype)], axis=0)
    hs2 = _tc_mid(acc1, xs, cnt3, W1p, b1.reshape(1, 32), W2)
    acc2 = _sc_spmm(
        hs2, s, d,
        ept=E // NC // NS, src_core_stride=E // NC, dst_core_stride=E // NC,
    ).reshape(NC, NACC, 16)

    return _tc_out(acc2, hs2, cnt3, b2.reshape(1, 16))
